# baseline (speedup 1.0000x reference)
import functools

import jax
import jax.numpy as jnp
from jax import lax
from jax.experimental import pallas as pl
from jax.experimental.pallas import tpu as pltpu
from jax.experimental.pallas import tpu_sc as plsc


@functools.lru_cache(maxsize=None)
def _build_lookup(B: int, T: int, V: int, D: int):
    info = plsc.get_sparse_core_info()
    NC, NS, L = info.num_cores, info.num_subcores, info.num_lanes
    NW = NC * NS
    SUB, LANE = 8, 128
    VP = (V + 7) // 8 * 8
    assert D == 2 * SUB and B % (NW * LANE) == 0 and T % SUB == 0
    TT = T // SUB
    BPW = B // (NW * LANE)
    UNITS = TT * BPW
    assert UNITS % 2 == 0
    K = LANE // L

    mesh = plsc.VectorSubcoreMesh(core_axis_name="c", subcore_axis_name="s")

    @functools.partial(
        pl.kernel,
        mesh=mesh,
        out_type=jax.ShapeDtypeStruct((T, D, B), jnp.float32),
        scratch_types=[
            pltpu.VMEM((VP * D,), jnp.float32),
            pltpu.VMEM((SUB, LANE), jnp.int32),
            pltpu.VMEM((SUB, LANE), jnp.int32),
            pltpu.VMEM((SUB, D, LANE), jnp.float32),
            pltpu.VMEM((SUB, D, LANE), jnp.float32),
            pltpu.SemaphoreType.DMA,
            pltpu.SemaphoreType.DMA,
            pltpu.SemaphoreType.DMA,
            pltpu.SemaphoreType.DMA,
        ],
        compiler_params=pltpu.CompilerParams(needs_layout_passes=False),
    )
    def lookup(seq_hbm, tab_hbm, out_hbm, tab_v, seq0, seq1, outb0, outb1,
               sem_i0, sem_i1, sem_o0, sem_o1):
        wid = lax.axis_index("s") * NC + lax.axis_index("c")
        bbase = wid * (BPW * LANE)

        pltpu.sync_copy(tab_hbm, tab_v)

        def unit_pos(u):
            t0 = (u // BPW) * SUB
            b0 = bbase + (u % BPW) * LANE
            return t0, b0

        def start_in(u, seq_b, sem_i):
            t0, b0 = unit_pos(u)
            pltpu.async_copy(
                seq_hbm.at[pl.ds(t0, SUB), pl.ds(b0, LANE)], seq_b, sem_i)

        start_in(0, seq0, sem_i0)
        start_in(1, seq1, sem_i1)

        def expand(seq_b, out_b):
            for t in range(SUB):
                for k2 in range(K // 2):
                    cols = []
                    for k in (2 * k2, 2 * k2 + 1):
                        sv = seq_b[t, pl.ds(k * L, L)]
                        cols.append([
                            plsc.load_gather(
                                tab_v.at[pl.ds(d * VP, V)], [sv])
                            for d in range(D)])
                    for i, k in enumerate((2 * k2, 2 * k2 + 1)):
                        for d in range(D):
                            out_b[t, d, pl.ds(k * L, L)] = cols[i][d]

        def half(u, seq_b, out_b, sem_i, sem_o):
            pltpu.make_async_copy(
                seq_hbm.at[pl.ds(0, SUB), pl.ds(0, LANE)], seq_b, sem_i).wait()

            @pl.when(u >= 2)
            def _():
                pltpu.make_async_copy(
                    out_b, out_hbm.at[pl.ds(0, SUB), pl.ds(0, D),
                                      pl.ds(0, LANE)], sem_o).wait()

            expand(seq_b, out_b)
            t0, b0 = unit_pos(u)
            pltpu.async_copy(
                out_b,
                out_hbm.at[pl.ds(t0, SUB), pl.ds(0, D), pl.ds(b0, LANE)],
                sem_o)

            @pl.when(u + 2 < UNITS)
            def _():
                start_in(u + 2, seq_b, sem_i)

        def pair(p, carry):
            half(2 * p, seq0, outb0, sem_i0, sem_o0)
            half(2 * p + 1, seq1, outb1, sem_i1, sem_o1)
            return carry

        lax.fori_loop(0, UNITS // 2, pair, 0)
        pltpu.make_async_copy(
            outb0, out_hbm.at[pl.ds(0, SUB), pl.ds(0, D), pl.ds(0, LANE)],
            sem_o0).wait()
        pltpu.make_async_copy(
            outb1, out_hbm.at[pl.ds(0, SUB), pl.ds(0, D), pl.ds(0, LANE)],
            sem_o1).wait()

    return lookup


def kernel(seq, emb_weight):
    B, T = seq.shape
    V, D = emb_weight.shape
    seq_t = jnp.transpose(seq).astype(jnp.int32)
    VP = (V + 7) // 8 * 8
    tab_cm = jnp.pad(jnp.transpose(emb_weight), ((0, 0), (0, VP - V))
                     ).reshape(D * VP)
    out_t = _build_lookup(B, T, V, D)(seq_t, tab_cm)
    return jnp.transpose(out_t, (2, 0, 1))


# speedup vs baseline: 3.6188x; 3.6188x over previous
import functools

import jax
import jax.numpy as jnp
from jax import lax
from jax.experimental import pallas as pl
from jax.experimental.pallas import tpu as pltpu
from jax.experimental.pallas import tpu_sc as plsc


@functools.lru_cache(maxsize=None)
def _build_lookup(B: int, T: int, V: int, D: int):
    info = plsc.get_sparse_core_info()
    NC, NS, L = info.num_cores, info.num_subcores, info.num_lanes
    NW = NC * NS
    SUB, LANE = 8, 128
    VP = (V + 7) // 8 * 8
    assert D == 2 * SUB and B % (NW * LANE) == 0 and T % SUB == 0
    TT = T // SUB
    BPW = B // (NW * LANE)
    UNITS = TT * BPW
    assert UNITS % 2 == 0
    K = LANE // L

    mesh = plsc.VectorSubcoreMesh(core_axis_name="c", subcore_axis_name="s")

    @functools.partial(
        pl.kernel,
        mesh=mesh,
        out_type=jax.ShapeDtypeStruct((T, D, B), jnp.float32),
        scratch_types=[
            pltpu.VMEM((VP * D,), jnp.float32),
            pltpu.VMEM((SUB, LANE), jnp.int32),
            pltpu.VMEM((SUB, LANE), jnp.int32),
            pltpu.VMEM((SUB, D, LANE), jnp.float32),
            pltpu.VMEM((SUB, D, LANE), jnp.float32),
            pltpu.SemaphoreType.DMA,
            pltpu.SemaphoreType.DMA,
            pltpu.SemaphoreType.DMA,
            pltpu.SemaphoreType.DMA,
        ],
        compiler_params=pltpu.CompilerParams(needs_layout_passes=False),
    )
    def lookup(seq_hbm, tab_hbm, out_hbm, tab_v, seq0, seq1, outb0, outb1,
               sem_i0, sem_i1, sem_o0, sem_o1):
        wid = lax.axis_index("s") * NC + lax.axis_index("c")
        bbase = wid * (BPW * LANE)

        pltpu.sync_copy(tab_hbm, tab_v)

        def unit_pos(u):
            t0 = (u // BPW) * SUB
            b0 = bbase + (u % BPW) * LANE
            return t0, b0

        def start_in(u, seq_b, sem_i):
            t0, b0 = unit_pos(u)
            pltpu.async_copy(
                seq_hbm.at[pl.ds(t0, SUB), pl.ds(b0, LANE)], seq_b, sem_i)

        start_in(0, seq0, sem_i0)
        start_in(1, seq1, sem_i1)

        def expand(seq_b, out_b):
            @functools.partial(plsc.parallel_loop, 0, SUB * K, unroll=2)
            def _(g):
                t = g // K
                k = g % K
                sv = seq_b[t, pl.ds(k * L, L)]
                cols = [plsc.load_gather(tab_v.at[pl.ds(d * VP, V)], [sv])
                        for d in range(D)]
                for d in range(D):
                    out_b[t, d, pl.ds(k * L, L)] = cols[d]

        def half(u, seq_b, out_b, sem_i, sem_o):
            pltpu.make_async_copy(
                seq_hbm.at[pl.ds(0, SUB), pl.ds(0, LANE)], seq_b, sem_i).wait()

            @pl.when(u >= 2)
            def _():
                pltpu.make_async_copy(
                    out_b, out_hbm.at[pl.ds(0, SUB), pl.ds(0, D),
                                      pl.ds(0, LANE)], sem_o).wait()

            expand(seq_b, out_b)
            t0, b0 = unit_pos(u)
            pltpu.async_copy(
                out_b,
                out_hbm.at[pl.ds(t0, SUB), pl.ds(0, D), pl.ds(b0, LANE)],
                sem_o)

            @pl.when(u + 2 < UNITS)
            def _():
                start_in(u + 2, seq_b, sem_i)

        def pair(p, carry):
            half(2 * p, seq0, outb0, sem_i0, sem_o0)
            half(2 * p + 1, seq1, outb1, sem_i1, sem_o1)
            return carry

        lax.fori_loop(0, UNITS // 2, pair, 0)
        pltpu.make_async_copy(
            outb0, out_hbm.at[pl.ds(0, SUB), pl.ds(0, D), pl.ds(0, LANE)],
            sem_o0).wait()
        pltpu.make_async_copy(
            outb1, out_hbm.at[pl.ds(0, SUB), pl.ds(0, D), pl.ds(0, LANE)],
            sem_o1).wait()

    return lookup


def kernel(seq, emb_weight):
    B, T = seq.shape
    V, D = emb_weight.shape
    seq_t = jnp.transpose(seq).astype(jnp.int32)
    VP = (V + 7) // 8 * 8
    tab_cm = jnp.pad(jnp.transpose(emb_weight), ((0, 0), (0, VP - V))
                     ).reshape(D * VP)
    out_t = _build_lookup(B, T, V, D)(seq_t, tab_cm)
    return jnp.transpose(out_t, (2, 0, 1))
